# baseline (speedup 1.0000x reference)
import functools

import jax
import jax.numpy as jnp
from jax import lax
from jax.experimental import pallas as pl
from jax.experimental.pallas import tpu as pltpu
from jax.experimental.pallas import tpu_sc as plsc

NC = 2
NS = 16
NW = NC * NS

G_STATIC = 128


def _fill_vec(buf, vec, start, nv):
  def body(i, carry):
    buf[pl.ds(i * 16, 16)] = vec
    return carry
  lax.fori_loop(start, nv, body, 0)


def _sc_impl(E_BASE, E_EXP, N, ei, ea, bv, ee, wfv, wvn, aux16):
  NV = 4
  ATTR_BASE_W = E_BASE * 16
  ATTR_W_PER = ATTR_BASE_W // NW
  EXP_W_PER = E_EXP * 16 // NW
  IDX_PER = E_BASE // NW
  PAT = 20000
  RING = 20000
  IRING = 12504
  ICHUNKS = ((0, 12504), (12504, 12504), (25008, 12504), (37512, 12488))
  T0_PER = E_BASE // NW
  T1_PER = E_EXP // NW
  NT2 = 2 * NV * N
  T2_CHUNK = ((NT2 + NW - 1) // NW + 7) // 8 * 8
  T2_NFULL = NT2 // T2_CHUNK
  T2_REM = NT2 - T2_NFULL * T2_CHUNK
  PAIRS_PER = E_EXP // NW
  DROUNDS = [(k * 3200, 3200, 200) for k in range(7)] + [(22400, 2600, 163)]
  VEI_SUB0 = ((0, 12496, 781), (12496, 12496, 781))
  VEI_SUB1 = ((24992, 12496, 781), (37488, 12512, 782))
  VH_W = G_STATIC * 128

  O_IDX = 2 * (E_BASE + E_EXP)
  O_ATTR = (E_BASE + E_EXP) * 16
  O_TYPES = E_BASE + E_EXP + 2 * NV * N
  O_VEI = 2 * (2 * NV * N)
  O_VATTR = 2 * NV * N * 16

  mesh = plsc.VectorSubcoreMesh(
      core_axis_name="c", subcore_axis_name="s", num_cores=NC, num_subcores=NS)

  @functools.partial(
      pl.kernel,
      out_type=(
          jax.ShapeDtypeStruct((O_IDX,), jnp.int32),
          jax.ShapeDtypeStruct((O_ATTR,), jnp.float32),
          jax.ShapeDtypeStruct((O_TYPES,), jnp.int32),
          jax.ShapeDtypeStruct((NV * VH_W,), jnp.float32),
          jax.ShapeDtypeStruct((O_VEI,), jnp.int32),
          jax.ShapeDtypeStruct((O_VATTR,), jnp.float32),
      ),
      mesh=mesh,
      compiler_params=pltpu.CompilerParams(needs_layout_passes=False),
      scratch_types=[
          pltpu.VMEM((PAT,), jnp.float32),
          pltpu.VMEM((12000,), jnp.int32),
          pltpu.VMEM((IRING + 12,), jnp.int32),
          pltpu.VMEM((3200,), jnp.int32),
          pltpu.VMEM((3200,), jnp.int32),
          pltpu.VMEM((IRING + 12,), jnp.int32),
          pltpu.VMEM((RING,), jnp.float32),
          pltpu.VMEM((RING,), jnp.float32),
          pltpu.VMEM((16,), jnp.int32),
          pltpu.VMEM((144,), jnp.float32),
          pltpu.SemaphoreType.DMA,
          pltpu.SemaphoreType.DMA,
          pltpu.SemaphoreType.DMA,
          pltpu.SemaphoreType.DMA,
          pltpu.SemaphoreType.DMA,
          pltpu.SemaphoreType.DMA,
      ],
  )
  def body(ei, ea, bv, ee, wfv, wvn, aux16,
           o_idx, o_attr, o_types, o_vh, o_vei, o_vattr,
           pat_f, cbuf, dbin, dbs, dbd, vbuf, bb0, bb1, auxv, wfv_v,
           sem_pat, sem_bg, sem_d, sem_v, sem_in, sem_out):
    wid = lax.axis_index("s") * NC + lax.axis_index("c")
    drain = []

    pltpu.sync_copy(wfv, wfv_v)
    frow = wid // 2
    fh = wid % 2
    fvec = plsc.load_gather(wfv_v, [jnp.zeros((16,), jnp.int32) + frow])
    _fill_vec(pat_f, fvec, 0, PAT // 16)
    p2_outs = []
    EXP_HALF = E_EXP // 2
    for j in range(EXP_HALF // PAT):
      off = frow * (E_BASE + E_EXP) + E_BASE + fh * EXP_HALF + j * PAT
      p2_outs.append(
          pltpu.async_copy(pat_f, o_attr.at[pl.ds(off, PAT)], sem_pat))

    _fill_vec(cbuf, jnp.zeros((16,), jnp.int32), 0, 375)
    _fill_vec(cbuf, jnp.full((16,), 1, jnp.int32), 375, 625)
    _fill_vec(cbuf, jnp.full((16,), 2, jnp.int32), 625, 750)
    base = wid * T0_PER
    for j in range(8):
      drain.append(pltpu.async_copy(
          cbuf.at[pl.ds(0, 6000)],
          o_types.at[pl.ds(base + j * 6000, 6000)], sem_bg))
    drain.append(pltpu.async_copy(
        cbuf.at[pl.ds(0, 2000)], o_types.at[pl.ds(base + 48000, 2000)], sem_bg))
    base = E_BASE + wid * T1_PER
    for j in range(6):
      drain.append(pltpu.async_copy(
          cbuf.at[pl.ds(6000, 4000)],
          o_types.at[pl.ds(base + j * 4000, 4000)], sem_bg))
    drain.append(pltpu.async_copy(
        cbuf.at[pl.ds(6000, 1000)],
        o_types.at[pl.ds(base + 24000, 1000)], sem_bg))
    base = E_BASE + E_EXP

    @pl.when(wid < T2_NFULL)
    def _():
      b2 = base + wid * T2_CHUNK
      for j in range(6):
        pltpu.async_copy(cbuf.at[pl.ds(10000, 2000)],
                         o_types.at[pl.ds(b2 + j * 2000, 2000)], sem_bg).wait()
      pltpu.async_copy(cbuf.at[pl.ds(10000, T2_CHUNK - 12000)],
                       o_types.at[pl.ds(b2 + 12000, T2_CHUNK - 12000)],
                       sem_bg).wait()

    @pl.when(wid == T2_NFULL)
    def _():
      b2 = base + T2_NFULL * T2_CHUNK
      for j in range(6):
        pltpu.async_copy(cbuf.at[pl.ds(10000, 2000)],
                         o_types.at[pl.ds(b2 + j * 2000, 2000)], sem_bg).wait()
      pltpu.async_copy(cbuf.at[pl.ds(10000, T2_REM - 12000)],
                       o_types.at[pl.ds(b2 + 12000, T2_REM - 12000)],
                       sem_bg).wait()

    iota2 = lax.iota(jnp.int32, 16) * 2
    pbase = wid * PAIRS_PER
    prev = []
    for (poff, npairs, nvregs) in DROUNDS:
      pltpu.sync_copy(ee.at[pl.ds((pbase + poff) * 2, npairs * 2)],
                      dbin.at[pl.ds(0, npairs * 2)])
      for d in prev:
        d.wait()
      prev = []

      def deint(j, carry):
        idx = iota2 + j * 32
        dbs[pl.ds(j * 16, 16)] = plsc.load_gather(dbin, [idx])
        dbd[pl.ds(j * 16, 16)] = plsc.load_gather(dbin, [idx + 1])
        return carry

      lax.fori_loop(0, nvregs, deint, 0)
      prev.append(pltpu.async_copy(
          dbs.at[pl.ds(0, npairs)],
          o_idx.at[pl.ds(E_BASE + pbase + poff, npairs)], sem_d))
      prev.append(pltpu.async_copy(
          dbd.at[pl.ds(0, npairs)],
          o_idx.at[pl.ds(2 * E_BASE + E_EXP + pbase + poff, npairs)], sem_d))
    for d in prev:
      d.wait()

    pltpu.sync_copy(aux16, auxv)
    p = wid // 2
    h = wid % 2
    off_iota = jnp.where(p < 4, p * 2 * N, 2 * NV * N + (2 * p - 7) * N)
    j = p - 8
    off_bv = jnp.where(j < 4, (2 * j + 1) * N, 2 * NV * N + (2 * j - 8) * N)
    k = jnp.maximum(j, 0) % 4

    def gen_iota(sub, out_off):
      for (hoff, sz, nv) in sub:
        def fill(i, carry):
          vbuf[pl.ds(i * 16, 16)] = lax.iota(jnp.int32, 16) + (hoff + i * 16)
          return carry
        lax.fori_loop(0, nv, fill, 0)
        pltpu.async_copy(vbuf.at[pl.ds(0, sz)],
                         o_vei.at[pl.ds(out_off + hoff, sz)], sem_v).wait()

    def gen_bv(sub, out_off):
      cvec = plsc.load_gather(auxv, [jnp.zeros((16,), jnp.int32) + k])
      for (hoff, sz, nv) in sub:
        pltpu.sync_copy(bv.at[pl.ds(hoff, sz)], vbuf.at[pl.ds(0, sz)])

        def addc(i, carry):
          vbuf[pl.ds(i * 16, 16)] = vbuf[pl.ds(i * 16, 16)] + cvec
          return carry
        lax.fori_loop(0, nv, addc, 0)
        pltpu.async_copy(vbuf.at[pl.ds(0, sz)],
                         o_vei.at[pl.ds(out_off + hoff, sz)], sem_v).wait()

    @pl.when((p < 8) & (h == 0))
    def _():
      gen_iota(VEI_SUB0, off_iota)

    @pl.when((p < 8) & (h == 1))
    def _():
      gen_iota(VEI_SUB1, off_iota)

    @pl.when((p >= 8) & (h == 0))
    def _():
      gen_bv(VEI_SUB0, off_bv)

    @pl.when((p >= 8) & (h == 1))
    def _():
      gen_bv(VEI_SUB1, off_bv)

    for d in p2_outs:
      d.wait()
    p7_outs = []
    for i in range(4):
      b = wid * 4 + i
      fr = b // 8
      sg = b % 8
      val = plsc.load_gather(
          wfv_v, [jnp.full((16,), 16, jnp.int32) + sg * 16 + fr])
      for d in p7_outs:
        d.wait()
      p7_outs = []
      _fill_vec(pat_f, val, 0, PAT // 16)
      off = fr * (2 * NV * N) + sg * N
      p7_outs.append(
          pltpu.async_copy(pat_f, o_vattr.at[pl.ds(off, PAT)], sem_pat))
      p7_outs.append(
          pltpu.async_copy(pat_f, o_vattr.at[pl.ds(off + PAT, PAT)], sem_pat))
      p7_outs.append(pltpu.async_copy(
          pat_f.at[pl.ds(0, N - 2 * PAT)],
          o_vattr.at[pl.ds(off + 2 * PAT, N - 2 * PAT)], sem_pat))

    for d in p7_outs:
      d.wait()

    @pl.when(wid < NV)
    def _():
      pltpu.sync_copy(wvn.at[pl.ds(wid * 128, 128)], pat_f.at[pl.ds(0, 128)])
      vs = [pat_f[pl.ds(r * 16, 16)] for r in range(8)]

      def repl(i, carry):
        for r in range(8):
          pat_f[pl.ds(i * 128 + r * 16, 16)] = vs[r]
        return carry

      lax.fori_loop(1, VH_W // 128, repl, 0)
      pltpu.async_copy(pat_f.at[pl.ds(0, VH_W)],
                       o_vh.at[pl.ds(wid * VH_W, VH_W)], sem_pat).wait()

    ijobs = []
    for r in range(2):
      for (coff, csz) in ICHUNKS:
        ijobs.append((E_BASE * r + wid * IDX_PER + coff,
                      (E_BASE + E_EXP) * r + wid * IDX_PER + coff, csz))
    ibbs = [dbin, vbuf]
    ni = len(ijobs)
    i_in = [None] * ni
    i_out = [None] * ni

    def istart(i):
      soff, _, csz = ijobs[i]
      return pltpu.async_copy(ei.at[pl.ds(soff, csz)],
                              ibbs[i % 2].at[pl.ds(0, csz)], sem_in)

    i_in[0] = istart(0)
    for i in range(ni):
      if i + 1 < ni:
        if i >= 1:
          i_out[i - 1].wait()
        i_in[i + 1] = istart(i + 1)
      i_in[i].wait()
      _, doff, csz = ijobs[i]
      i_out[i] = pltpu.async_copy(ibbs[i % 2].at[pl.ds(0, csz)],
                                  o_idx.at[pl.ds(doff, csz)], sem_out)
    i_out[ni - 2].wait()
    i_out[ni - 1].wait()

    ajobs = []
    for fr2 in range(16):
      for (coff, csz) in ((0, RING), (RING, RING), (2 * RING, IDX_PER - 2 * RING)):
        ajobs.append((fr2 * E_BASE + wid * IDX_PER + coff,
                      fr2 * (E_BASE + E_EXP) + wid * IDX_PER + coff, csz))
    bbs = [bb0, bb1]
    n = len(ajobs)

    def astart(i):
      soff, _, csz = ajobs[i]
      return pltpu.async_copy(ea.at[pl.ds(soff, csz)],
                              bbs[i % 2].at[pl.ds(0, csz)], sem_in)

    a_out = [None] * n
    a_in = [None] * n
    a_in[0] = astart(0)
    for i in range(n):
      if i + 1 < n:
        if i >= 1:
          a_out[i - 1].wait()
        a_in[i + 1] = astart(i + 1)
      a_in[i].wait()
      _, doff, csz = ajobs[i]
      a_out[i] = pltpu.async_copy(bbs[i % 2].at[pl.ds(0, csz)],
                                  o_attr.at[pl.ds(doff, csz)], sem_out)
    a_out[n - 2].wait()
    a_out[n - 1].wait()

    for d in drain:
      d.wait()

  return body(ei, ea, bv, ee, wfv, wvn, aux16)


def kernel(edge_index, edge_attr, batch_vec, expander_edges, num_graphs,
           exp_edge_attr_weight, virt_node_emb_weight,
           virt_edge_in_emb_weight, virt_edge_out_emb_weight):
  E_BASE = edge_index.shape[1]
  E_EXP = expander_edges.shape[0]
  N = batch_vec.shape[0]
  NV = virt_node_emb_weight.shape[0]

  ei = edge_index.reshape(-1)
  ea = edge_attr.T.reshape(-1)
  ee = expander_edges.reshape(-1)
  wseg = jnp.stack(
      [virt_edge_in_emb_weight, virt_edge_out_emb_weight], axis=1).reshape(-1)
  wfv = jnp.concatenate([exp_edge_attr_weight.reshape(-1), wseg])
  wvn = virt_node_emb_weight.reshape(-1)
  c4 = N + jnp.arange(NV, dtype=jnp.int32) * num_graphs
  aux16 = jnp.concatenate([c4, jnp.zeros((16 - NV,), jnp.int32)])

  o_idx, o_attr, o_types, o_vh, o_vei, o_vattr = _sc_impl(
      E_BASE, E_EXP, N, ei, ea, batch_vec, ee, wfv, wvn, aux16)

  return (
      o_idx.reshape(2, E_BASE + E_EXP),
      o_attr.reshape(16, E_BASE + E_EXP).T,
      o_types,
      o_vh.reshape(NV * G_STATIC, 128),
      o_vei.reshape(2, 2 * NV * N),
      o_vattr.reshape(16, 2 * NV * N).T,
  )


# speedup vs baseline: 8.6055x; 8.6055x over previous
import functools

import jax
import jax.numpy as jnp
from jax import lax
from jax.experimental import pallas as pl
from jax.experimental.pallas import tpu as pltpu
from jax.experimental.pallas import tpu_sc as plsc

NC = 2
NS = 16
NW = NC * NS

G_STATIC = 128


def _sc_impl(E_BASE, E_EXP, N, ei, ea, bv, ee, wfv, wvn, aux16):
  NV = 4
  CB_BASE = E_BASE // 128
  CB_ALL = (E_BASE + E_EXP) // 128
  CB_EXP = CB_ALL - CB_BASE
  CB_VEI = 2 * NV * N // 128
  PAT = 19456
  RING = 8000
  IRING = 10000
  T0_PER = E_BASE // NW
  T1_PER = E_EXP // NW
  NT2 = 2 * NV * N
  T2_CHUNK = ((NT2 + NW - 1) // NW + 7) // 8 * 8
  T2_NFULL = NT2 // T2_CHUNK
  T2_REM = NT2 - T2_NFULL * T2_CHUNK
  VH_W = G_STATIC * 128

  O_IDX = 2 * (E_BASE + E_EXP)
  O_ATTR = (E_BASE + E_EXP) * 16
  O_TYPES = E_BASE + E_EXP + 2 * NV * N
  O_VEI = 2 * (2 * NV * N)
  O_VATTR = 2 * NV * N * 16

  mesh = plsc.VectorSubcoreMesh(
      core_axis_name="c", subcore_axis_name="s", num_cores=NC, num_subcores=NS)

  @functools.partial(
      pl.kernel,
      out_type=(
          jax.ShapeDtypeStruct((O_IDX,), jnp.int32),
          jax.ShapeDtypeStruct((O_ATTR,), jnp.float32),
          jax.ShapeDtypeStruct((O_TYPES,), jnp.int32),
          jax.ShapeDtypeStruct((NV * VH_W,), jnp.float32),
          jax.ShapeDtypeStruct((O_VEI,), jnp.int32),
          jax.ShapeDtypeStruct((O_VATTR,), jnp.float32),
      ),
      mesh=mesh,
      compiler_params=pltpu.CompilerParams(needs_layout_passes=False),
      scratch_types=[
          pltpu.VMEM((PAT,), jnp.float32),
          pltpu.VMEM((6000,), jnp.int32),
          pltpu.VMEM((50000,), jnp.int32),
          pltpu.VMEM((25088,), jnp.int32),
          pltpu.VMEM((12544,), jnp.int32),
          pltpu.VMEM((RING,), jnp.float32),
          pltpu.VMEM((RING,), jnp.float32),
          pltpu.VMEM((16,), jnp.int32),
          pltpu.VMEM((144,), jnp.float32),
          pltpu.SemaphoreType.DMA,
          pltpu.SemaphoreType.DMA,
          pltpu.SemaphoreType.DMA,
          pltpu.SemaphoreType.DMA,
          pltpu.SemaphoreType.DMA,
          pltpu.SemaphoreType.DMA,
          pltpu.SemaphoreType.DMA,
          pltpu.SemaphoreType.DMA,
      ],
  )
  def body(ei, ea, bv, ee, wfv, wvn, aux16,
           o_idx, o_attr, o_types, o_vh, o_vei, o_vattr,
           pat_f, cbuf, bv_v, stage, db, bb0, bb1, auxv, wfv_v,
           sem_pat, sem_bg, sem_st, sem_st2, sem_in0, sem_in1,
           sem_out0, sem_out1):
    wid = lax.axis_index("s") * NC + lax.axis_index("c")
    drain = []
    iota = lax.iota(jnp.int32, 16)
    zeros16 = jnp.zeros((16,), jnp.int32)

    pltpu.sync_copy(wfv, wfv_v)
    pltpu.sync_copy(aux16, auxv)

    rb = wid % 2
    w = wid // 2
    for r in range(8):
      vec = plsc.load_gather(wfv_v, [zeros16 + (rb * 8 + r)])

      def fillp(t, carry):
        for v in range(8):
          pat_f[pl.ds(t * 1024 + r * 128 + v * 16, 16)] = vec
        return carry
      lax.fori_loop(0, PAT // 1024, fillp, 0)
    exp_w0 = rb * CB_ALL * 1024 + CB_BASE * 1024
    tile0 = jnp.where(w < 10, w * 391, 3910 + (w - 10) * 390)
    off = exp_w0 + tile0 * 1024
    p2_outs = []
    for j in range(20):
      p2_outs.append(pltpu.async_copy(
          pat_f, o_attr.at[pl.ds(off + j * PAT, PAT)], sem_pat))

    @pl.when(w < 10)
    def _():
      pltpu.async_copy(pat_f.at[pl.ds(0, 11264)],
                       o_attr.at[pl.ds(off + 20 * PAT, 11264)],
                       sem_pat).wait()

    @pl.when(w >= 10)
    def _():
      pltpu.async_copy(pat_f.at[pl.ds(0, 10240)],
                       o_attr.at[pl.ds(off + 20 * PAT, 10240)],
                       sem_pat).wait()

    def cfill(buf, val, start, nv):
      vec = zeros16 + val

      def bodyf(i, carry):
        buf[pl.ds(i * 16, 16)] = vec
        return carry
      lax.fori_loop(start, nv, bodyf, 0)

    cfill(cbuf, 0, 0, 188)
    cfill(cbuf, 1, 188, 313)
    cfill(cbuf, 2, 313, 375)
    base = wid * T0_PER
    for j in range(16):
      drain.append(pltpu.async_copy(
          cbuf.at[pl.ds(0, 3008)],
          o_types.at[pl.ds(base + j * 3008, 3008)], sem_bg))
    drain.append(pltpu.async_copy(
        cbuf.at[pl.ds(0, 1872)], o_types.at[pl.ds(base + 48128, 1872)], sem_bg))
    base = E_BASE + wid * T1_PER
    for j in range(12):
      drain.append(pltpu.async_copy(
          cbuf.at[pl.ds(3008, 2000)],
          o_types.at[pl.ds(base + j * 2000, 2000)], sem_bg))
    drain.append(pltpu.async_copy(
        cbuf.at[pl.ds(3008, 1000)],
        o_types.at[pl.ds(base + 24000, 1000)], sem_bg))
    base = E_BASE + E_EXP

    @pl.when(wid < T2_NFULL)
    def _():
      b2 = base + wid * T2_CHUNK
      for j in range(12):
        pltpu.async_copy(cbuf.at[pl.ds(5008, 992)],
                         o_types.at[pl.ds(b2 + j * 992, 992)], sem_bg).wait()
      pltpu.async_copy(cbuf.at[pl.ds(5008, T2_CHUNK - 11904)],
                       o_types.at[pl.ds(b2 + 11904, T2_CHUNK - 11904)],
                       sem_bg).wait()

    @pl.when(wid == T2_NFULL)
    def _():
      b2 = base + T2_NFULL * T2_CHUNK
      for j in range(12):
        pltpu.async_copy(cbuf.at[pl.ds(5008, 992)],
                         o_types.at[pl.ds(b2 + j * 992, 992)], sem_bg).wait()
      pltpu.async_copy(cbuf.at[pl.ds(5008, T2_REM - 11904)],
                       o_types.at[pl.ds(b2 + 11904, T2_REM - 11904)],
                       sem_bg).wait()

    pltpu.sync_copy(bv, bv_v)
    vb0 = jnp.where(wid < 21, wid * 98, 2058 + (wid - 21) * 97)

    def vei_block(b, carry):
      cb = vb0 + b
      for v in range(8):
        colv = iota + (cb * 128 + v * 16)
        s = colv // N
        il = colv - s * N
        bvv = plsc.load_gather(bv_v, [il])
        cv = plsc.load_gather(auxv, [s >> 1])
        vt = bvv + cv
        evn = (s & 1) == 0
        stage[pl.ds(b * 256 + v * 16, 16)] = jnp.where(evn, il, vt)
        stage[pl.ds(b * 256 + 128 + v * 16, 16)] = jnp.where(evn, vt, il)
      return carry

    nblk = jnp.where(wid < 21, 98, 97)
    lax.fori_loop(0, nblk, vei_block, 0)

    @pl.when(wid < 21)
    def _():
      pltpu.async_copy(stage.at[pl.ds(0, 98 * 256)],
                       o_vei.at[pl.ds(vb0 * 256, 98 * 256)], sem_st).wait()

    @pl.when(wid >= 21)
    def _():
      pltpu.async_copy(stage.at[pl.ds(0, 97 * 256)],
                       o_vei.at[pl.ds(vb0 * 256, 97 * 256)], sem_st).wait()

    ib0 = jnp.where(wid < 10, wid * 196, 1960 + (wid - 10) * 195)

    def idx_round(blk0, nb):
      pltpu.sync_copy(ee.at[pl.ds(blk0 * 128, nb * 128)],
                      db.at[pl.ds(0, nb * 128)])
      pltpu.sync_copy(ee.at[pl.ds(E_EXP + blk0 * 128, nb * 128)],
                      db.at[pl.ds(6272, nb * 128)])

      def inter(b, carry):
        for v in range(8):
          stage[pl.ds(b * 256 + v * 16, 16)] = db[pl.ds(b * 128 + v * 16, 16)]
          stage[pl.ds(b * 256 + 128 + v * 16, 16)] = (
              db[pl.ds(6272 + b * 128 + v * 16, 16)])
        return carry

      lax.fori_loop(0, nb, inter, 0)
      pltpu.async_copy(
          stage.at[pl.ds(0, nb * 256)],
          o_idx.at[pl.ds((CB_BASE + blk0) * 256, nb * 256)], sem_st).wait()

    for rr in range(3):
      idx_round(ib0 + rr * 49, 49)

    @pl.when(wid < 10)
    def _():
      idx_round(ib0 + 147, 49)

    @pl.when(wid >= 10)
    def _():
      idx_round(ib0 + 147, 48)

    for d in p2_outs:
      d.wait()
    va0 = jnp.where(wid < 10, wid * 196, 1960 + (wid - 10) * 195)
    vhalves = [0, 9728]

    def wait_words(sem, nwords):
      pltpu.make_async_copy(o_vattr.at[pl.ds(0, nwords)],
                            pat_f.at[pl.ds(0, nwords)], sem).wait()

    def va_fill(rnd, t0, nt):
      half_off = (rnd % 2) * 9728

      def vat(tr, carry):
        t = tr // 8
        r = tr % 8
        T = t0 + t
        rb2 = T // CB_VEI
        cb2 = T % CB_VEI
        for v in range(8):
          colv = iota + (cb2 * 128 + v * 16)
          s = colv // N
          val = plsc.load_gather(wfv_v, [s * 16 + (16 + rb2 * 8 + r)])
          pat_f[pl.ds(half_off + t * 1024 + r * 128 + v * 16, 16)] = val
        return carry

      lax.fori_loop(0, nt * 8, vat, 0)
      return half_off

    def va_body(rr, carry):
      t0 = va0 + rr * 9

      @pl.when(rr >= 2)
      def _():
        @pl.when(rr % 2 == 0)
        def _():
          wait_words(sem_st, 9216)

        @pl.when(rr % 2 == 1)
        def _():
          wait_words(sem_st2, 9216)

      half_off = (rr % 2) * 9728

      def vat(tr, carry2):
        t = tr // 8
        r = tr % 8
        T = t0 + t
        rb2 = T // CB_VEI
        cb2 = T % CB_VEI
        for v in range(8):
          colv = iota + (cb2 * 128 + v * 16)
          s = colv // N
          val = plsc.load_gather(wfv_v, [s * 16 + (16 + rb2 * 8 + r)])
          pat_f[pl.ds(half_off + t * 1024 + r * 128 + v * 16, 16)] = val
        return carry2

      lax.fori_loop(0, 72, vat, 0)

      @pl.when(rr % 2 == 0)
      def _():
        pltpu.async_copy(pat_f.at[pl.ds(0, 9216)],
                         o_vattr.at[pl.ds(t0 * 1024, 9216)], sem_st)

      @pl.when(rr % 2 == 1)
      def _():
        pltpu.async_copy(pat_f.at[pl.ds(9728, 9216)],
                         o_vattr.at[pl.ds(t0 * 1024, 9216)], sem_st2)
      return carry

    lax.fori_loop(0, 21, va_body, 0)
    wait_words(sem_st2, 9216)

    @pl.when(wid < 10)
    def _():
      ho = va_fill(1, va0 + 189, 7)
      pltpu.async_copy(pat_f.at[pl.ds(ho, 7 * 1024)],
                       o_vattr.at[pl.ds((va0 + 189) * 1024, 7 * 1024)],
                       sem_st2).wait()

    @pl.when(wid >= 10)
    def _():
      ho = va_fill(1, va0 + 189, 6)
      pltpu.async_copy(pat_f.at[pl.ds(ho, 6 * 1024)],
                       o_vattr.at[pl.ds((va0 + 189) * 1024, 6 * 1024)],
                       sem_st2).wait()

    wait_words(sem_st, 9216)

    @pl.when(wid < NV)
    def _():
      pltpu.sync_copy(wvn.at[pl.ds(wid * 128, 128)], pat_f.at[pl.ds(0, 128)])
      vs = [pat_f[pl.ds(r * 16, 16)] for r in range(8)]

      def repl(i, carry):
        for r in range(8):
          pat_f[pl.ds(i * 128 + r * 16, 16)] = vs[r]
        return carry

      lax.fori_loop(1, VH_W // 128, repl, 0)
      pltpu.async_copy(pat_f.at[pl.ds(0, VH_W)],
                       o_vh.at[pl.ds(wid * VH_W, VH_W)], sem_pat).wait()

    IDXW = 2 * E_BASE // NW
    ibbs = [db, stage]
    njobs = IDXW // IRING

    def istart(i):
      return pltpu.async_copy(
          ei.at[pl.ds(wid * IDXW + i * IRING, IRING)],
          ibbs[i % 2].at[pl.ds(0, IRING)],
          sem_in0 if i % 2 == 0 else sem_in1)

    i_in = [None] * njobs
    i_out = [None] * njobs
    i_in[0] = istart(0)
    for i in range(njobs):
      if i + 1 < njobs:
        if i >= 1:
          i_out[i - 1].wait()
        i_in[i + 1] = istart(i + 1)
      i_in[i].wait()
      i_out[i] = pltpu.async_copy(
          ibbs[i % 2].at[pl.ds(0, IRING)],
          o_idx.at[pl.ds(wid * IDXW + i * IRING, IRING)],
          sem_out0 if i % 2 == 0 else sem_out1)
    i_out[njobs - 2].wait()
    i_out[njobs - 1].wait()

    PERT = CB_BASE * 1024 // NW
    NCH = PERT // RING
    NJ = 2 * NCH

    def asrc(i):
      return (i // NCH) * (CB_BASE * 1024) + wid * PERT + (i % NCH) * RING

    def adst(i):
      return (i // NCH) * (CB_ALL * 1024) + wid * PERT + (i % NCH) * RING

    def await_ring(sem, buf):
      pltpu.make_async_copy(ea.at[pl.ds(0, RING)], buf, sem).wait()

    pltpu.async_copy(ea.at[pl.ds(asrc(0), RING)], bb0, sem_in0)

    def ring_body(i, carry):
      @pl.when(i + 1 < NJ)
      def _():
        @pl.when(i >= 1)
        def _():
          @pl.when((i + 1) % 2 == 0)
          def _():
            await_ring(sem_out0, bb0)

          @pl.when((i + 1) % 2 == 1)
          def _():
            await_ring(sem_out1, bb1)

        @pl.when((i + 1) % 2 == 0)
        def _():
          pltpu.async_copy(ea.at[pl.ds(asrc(i + 1), RING)], bb0, sem_in0)

        @pl.when((i + 1) % 2 == 1)
        def _():
          pltpu.async_copy(ea.at[pl.ds(asrc(i + 1), RING)], bb1, sem_in1)

      @pl.when(i % 2 == 0)
      def _():
        await_ring(sem_in0, bb0)
        pltpu.async_copy(bb0, o_attr.at[pl.ds(adst(i), RING)], sem_out0)

      @pl.when(i % 2 == 1)
      def _():
        await_ring(sem_in1, bb1)
        pltpu.async_copy(bb1, o_attr.at[pl.ds(adst(i), RING)], sem_out1)
      return carry

    lax.fori_loop(0, NJ, ring_body, 0)
    await_ring(sem_out0, bb0)
    await_ring(sem_out1, bb1)

    for d in drain:
      d.wait()

  return body(ei, ea, bv, ee, wfv, wvn, aux16)


def kernel(edge_index, edge_attr, batch_vec, expander_edges, num_graphs,
           exp_edge_attr_weight, virt_node_emb_weight,
           virt_edge_in_emb_weight, virt_edge_out_emb_weight):
  E_BASE = edge_index.shape[1]
  E_EXP = expander_edges.shape[0]
  N = batch_vec.shape[0]
  NV = virt_node_emb_weight.shape[0]

  ei = edge_index.reshape(2, E_BASE // 128, 128).transpose(1, 0, 2).reshape(-1)
  ea = (edge_attr.T.reshape(2, 8, E_BASE // 128, 128)
        .transpose(0, 2, 1, 3).reshape(-1))
  ee = expander_edges.T.reshape(-1)
  wseg = jnp.stack(
      [virt_edge_in_emb_weight, virt_edge_out_emb_weight], axis=1).reshape(-1)
  wfv = jnp.concatenate([exp_edge_attr_weight.reshape(-1), wseg])
  wvn = virt_node_emb_weight.reshape(-1)
  c4 = N + jnp.arange(NV, dtype=jnp.int32) * num_graphs
  aux16 = jnp.concatenate([c4, jnp.zeros((16 - NV,), jnp.int32)])

  o_idx, o_attr, o_types, o_vh, o_vei, o_vattr = _sc_impl(
      E_BASE, E_EXP, N, ei, ea, batch_vec, ee, wfv, wvn, aux16)

  E_ALL = E_BASE + E_EXP
  NVC = 2 * NV * N
  return (
      o_idx.reshape(E_ALL // 128, 2, 128).transpose(1, 0, 2).reshape(2, E_ALL),
      (o_attr.reshape(2, E_ALL // 128, 8, 128).transpose(0, 2, 1, 3)
       .reshape(16, E_ALL).T),
      o_types,
      o_vh.reshape(NV * G_STATIC, 128),
      o_vei.reshape(NVC // 128, 2, 128).transpose(1, 0, 2).reshape(2, NVC),
      (o_vattr.reshape(2, NVC // 128, 8, 128).transpose(0, 2, 1, 3)
       .reshape(16, NVC).T),
  )


# speedup vs baseline: 23.7436x; 2.7591x over previous
import functools

import jax
import jax.numpy as jnp
from jax import lax
from jax.experimental import pallas as pl
from jax.experimental.pallas import tpu as pltpu
from jax.experimental.pallas import tpu_sc as plsc

NC = 2
NS = 16
NW = NC * NS

G_STATIC = 128


def _sc_impl(E_BASE, E_EXP, N, ei, ea, bv, ee, wfv, wvn, aux16):
  NV = 4
  CB_BASE = E_BASE // 128
  CB_ALL = (E_BASE + E_EXP) // 128
  CB_EXP = CB_ALL - CB_BASE
  CB_VEI = 2 * NV * N // 128
  PAT = 19456
  RING = 10000
  IRING = 10000
  T0_PER = E_BASE // NW
  T1_PER = E_EXP // NW
  NT2 = 2 * NV * N
  T2_CHUNK = ((NT2 + NW - 1) // NW + 7) // 8 * 8
  T2_NFULL = NT2 // T2_CHUNK
  T2_REM = NT2 - T2_NFULL * T2_CHUNK
  VH_W = G_STATIC * 128

  O_IDX = 2 * (E_BASE + E_EXP)
  O_ATTR = (E_BASE + E_EXP) * 16
  O_TYPES = E_BASE + E_EXP + 2 * NV * N
  O_VEI = 2 * (2 * NV * N)
  O_VATTR = 2 * NV * N * 16

  mesh = plsc.VectorSubcoreMesh(
      core_axis_name="c", subcore_axis_name="s", num_cores=NC, num_subcores=NS)

  @functools.partial(
      pl.kernel,
      out_type=(
          jax.ShapeDtypeStruct((O_IDX,), jnp.int32),
          jax.ShapeDtypeStruct((O_ATTR,), jnp.float32),
          jax.ShapeDtypeStruct((O_TYPES,), jnp.int32),
          jax.ShapeDtypeStruct((NV * VH_W,), jnp.float32),
          jax.ShapeDtypeStruct((O_VEI,), jnp.int32),
          jax.ShapeDtypeStruct((O_VATTR,), jnp.float32),
      ),
      mesh=mesh,
      compiler_params=pltpu.CompilerParams(needs_layout_passes=False),
      scratch_types=[
          pltpu.VMEM((PAT,), jnp.float32),
          pltpu.VMEM((6000,), jnp.int32),
          pltpu.VMEM((50000,), jnp.int32),
          pltpu.VMEM((12544,), jnp.int32),
          pltpu.VMEM((12544,), jnp.int32),
          pltpu.VMEM((RING,), jnp.float32),
          pltpu.VMEM((RING,), jnp.float32),
          pltpu.VMEM((16,), jnp.int32),
          pltpu.VMEM((144,), jnp.float32),
          pltpu.SemaphoreType.DMA,
          pltpu.SemaphoreType.DMA,
          pltpu.SemaphoreType.DMA,
          pltpu.SemaphoreType.DMA,
          pltpu.SemaphoreType.DMA,
          pltpu.SemaphoreType.DMA,
          pltpu.SemaphoreType.DMA,
          pltpu.SemaphoreType.DMA,
      ],
  )
  def body(ei, ea, bv, ee, wfv, wvn, aux16,
           o_idx, o_attr, o_types, o_vh, o_vei, o_vattr,
           pat_f, cbuf, bv_v, stage, db, bb0, bb1, auxv, wfv_v,
           sem_pat, sem_bg, sem_st, sem_st2, sem_in0, sem_in1,
           sem_out0, sem_out1):
    wid = lax.axis_index("s") * NC + lax.axis_index("c")
    drain = []
    iota = lax.iota(jnp.int32, 16)
    zeros16 = jnp.zeros((16,), jnp.int32)

    pltpu.sync_copy(wfv, wfv_v)
    pltpu.sync_copy(aux16, auxv)

    rb = wid % 2
    w = wid // 2
    for r in range(8):
      vec = plsc.load_gather(wfv_v, [zeros16 + (rb * 8 + r)])

      def fillp(t, carry):
        for v in range(8):
          pat_f[pl.ds(t * 1024 + r * 128 + v * 16, 16)] = vec
        return carry
      lax.fori_loop(0, PAT // 1024, fillp, 0)
    exp_w0 = rb * CB_ALL * 1024 + CB_BASE * 1024
    tile0 = jnp.where(w < 10, w * 391, 3910 + (w - 10) * 390)
    off = exp_w0 + tile0 * 1024
    p2_outs = []
    for j in range(20):
      p2_outs.append(pltpu.async_copy(
          pat_f, o_attr.at[pl.ds(off + j * PAT, PAT)], sem_pat))

    @pl.when(w < 10)
    def _():
      pltpu.async_copy(pat_f.at[pl.ds(0, 11264)],
                       o_attr.at[pl.ds(off + 20 * PAT, 11264)],
                       sem_pat).wait()

    @pl.when(w >= 10)
    def _():
      pltpu.async_copy(pat_f.at[pl.ds(0, 10240)],
                       o_attr.at[pl.ds(off + 20 * PAT, 10240)],
                       sem_pat).wait()

    def cfill(buf, val, start, nv):
      vec = zeros16 + val

      def bodyf(i, carry):
        buf[pl.ds(i * 16, 16)] = vec
        return carry
      lax.fori_loop(start, nv, bodyf, 0)

    cfill(cbuf, 0, 0, 188)
    cfill(cbuf, 1, 188, 313)
    cfill(cbuf, 2, 313, 375)
    base = wid * T0_PER
    for j in range(16):
      drain.append(pltpu.async_copy(
          cbuf.at[pl.ds(0, 3008)],
          o_types.at[pl.ds(base + j * 3008, 3008)], sem_bg))
    drain.append(pltpu.async_copy(
        cbuf.at[pl.ds(0, 1872)], o_types.at[pl.ds(base + 48128, 1872)], sem_bg))
    base = E_BASE + wid * T1_PER
    for j in range(12):
      drain.append(pltpu.async_copy(
          cbuf.at[pl.ds(3008, 2000)],
          o_types.at[pl.ds(base + j * 2000, 2000)], sem_bg))
    drain.append(pltpu.async_copy(
        cbuf.at[pl.ds(3008, 1000)],
        o_types.at[pl.ds(base + 24000, 1000)], sem_bg))
    base = E_BASE + E_EXP

    def t2_copies(b2, tail):
      for j in range(12):
        pltpu.async_copy(cbuf.at[pl.ds(5008, 992)],
                         o_types.at[pl.ds(b2 + j * 992, 992)], sem_bg).wait()
      pltpu.async_copy(cbuf.at[pl.ds(5008, tail)],
                       o_types.at[pl.ds(b2 + 11904, tail)], sem_bg).wait()

    @pl.when(wid < T2_NFULL)
    def _():
      t2_copies(base + wid * T2_CHUNK, T2_CHUNK - 11904)

    @pl.when(wid == T2_NFULL)
    def _():
      t2_copies(base + T2_NFULL * T2_CHUNK, T2_REM - 11904)

    pltpu.sync_copy(bv, bv_v)
    vb0 = jnp.where(wid < 21, wid * 98, 2058 + (wid - 21) * 97)

    def make_vei_block(buf, cb0):
      def vei_block(b, carry):
        cb = cb0 + b
        for v in range(8):
          colv = iota + (cb * 128 + v * 16)
          s = colv // N
          il = colv - s * N
          bvv = plsc.load_gather(bv_v, [il])
          cv = plsc.load_gather(auxv, [s >> 1])
          vt = bvv + cv
          evn = (s & 1) == 0
          buf[pl.ds(b * 256 + v * 16, 16)] = jnp.where(evn, il, vt)
          buf[pl.ds(b * 256 + 128 + v * 16, 16)] = jnp.where(evn, vt, il)
        return carry
      return vei_block

    lax.fori_loop(0, 49, make_vei_block(stage, vb0), 0)
    vei_a = pltpu.async_copy(stage.at[pl.ds(0, 49 * 256)],
                             o_vei.at[pl.ds(vb0 * 256, 49 * 256)], sem_st)
    nblk = jnp.where(wid < 21, 98, 97)
    lax.fori_loop(0, nblk - 49, make_vei_block(db, vb0 + 49), 0)

    @pl.when(wid < 21)
    def _():
      pltpu.async_copy(db.at[pl.ds(0, 49 * 256)],
                       o_vei.at[pl.ds((vb0 + 49) * 256, 49 * 256)],
                       sem_st2).wait()

    @pl.when(wid >= 21)
    def _():
      pltpu.async_copy(db.at[pl.ds(0, 48 * 256)],
                       o_vei.at[pl.ds((vb0 + 49) * 256, 48 * 256)],
                       sem_st2).wait()
    vei_a.wait()

    ib0 = jnp.where(wid < 10, wid * 196, 1960 + (wid - 10) * 195)

    def idx_round(blk0, nb):
      pltpu.sync_copy(ee.at[pl.ds(blk0 * 128, nb * 128)],
                      db.at[pl.ds(0, nb * 128)])
      pltpu.sync_copy(ee.at[pl.ds(E_EXP + blk0 * 128, nb * 128)],
                      db.at[pl.ds(6272, nb * 128)])

      def inter(b, carry):
        for v in range(8):
          stage[pl.ds(b * 256 + v * 16, 16)] = db[pl.ds(b * 128 + v * 16, 16)]
          stage[pl.ds(b * 256 + 128 + v * 16, 16)] = (
              db[pl.ds(6272 + b * 128 + v * 16, 16)])
        return carry

      lax.fori_loop(0, nb, inter, 0)
      pltpu.async_copy(
          stage.at[pl.ds(0, nb * 256)],
          o_idx.at[pl.ds((CB_BASE + blk0) * 256, nb * 256)], sem_st).wait()

    for rr in range(3):
      idx_round(ib0 + rr * 49, 49)

    @pl.when(wid < 10)
    def _():
      idx_round(ib0 + 147, 49)

    @pl.when(wid >= 10)
    def _():
      idx_round(ib0 + 147, 48)

    for d in p2_outs:
      d.wait()
    va0 = jnp.where(wid < 10, wid * 196, 1960 + (wid - 10) * 195)
    vhalves = [0, 9728]

    def wait_words(sem, nwords):
      pltpu.make_async_copy(o_vattr.at[pl.ds(0, nwords)],
                            pat_f.at[pl.ds(0, nwords)], sem).wait()

    def va_fill(rnd, t0, nt):
      half_off = (rnd % 2) * 9728

      def vat(t, carry):
        T = t0 + t
        rb2 = T // CB_VEI
        cb2 = T % CB_VEI
        s0 = cb2 * 128 // N
        s127 = (cb2 * 128 + 127) // N

        @pl.when(s0 == s127)
        def _():
          for r in range(8):
            val = plsc.load_gather(
                wfv_v, [zeros16 + (s0 * 16 + 16 + rb2 * 8 + r)])
            for v in range(8):
              pat_f[pl.ds(half_off + t * 1024 + r * 128 + v * 16, 16)] = val

        @pl.when(s0 != s127)
        def _():
          for r in range(8):
            for v in range(8):
              colv = iota + (cb2 * 128 + v * 16)
              s = colv // N
              val = plsc.load_gather(wfv_v, [s * 16 + (16 + rb2 * 8 + r)])
              pat_f[pl.ds(half_off + t * 1024 + r * 128 + v * 16, 16)] = val
        return carry

      lax.fori_loop(0, nt, vat, 0)
      return half_off

    def va_body(rr, carry):
      t0 = va0 + rr * 9

      @pl.when(rr >= 2)
      def _():
        @pl.when(rr % 2 == 0)
        def _():
          wait_words(sem_st, 9216)

        @pl.when(rr % 2 == 1)
        def _():
          wait_words(sem_st2, 9216)

      va_fill(rr, t0, 9)

      @pl.when(rr % 2 == 0)
      def _():
        pltpu.async_copy(pat_f.at[pl.ds(0, 9216)],
                         o_vattr.at[pl.ds(t0 * 1024, 9216)], sem_st)

      @pl.when(rr % 2 == 1)
      def _():
        pltpu.async_copy(pat_f.at[pl.ds(9728, 9216)],
                         o_vattr.at[pl.ds(t0 * 1024, 9216)], sem_st2)
      return carry

    lax.fori_loop(0, 21, va_body, 0)
    wait_words(sem_st2, 9216)

    @pl.when(wid < 10)
    def _():
      ho = va_fill(1, va0 + 189, 7)
      pltpu.async_copy(pat_f.at[pl.ds(ho, 7 * 1024)],
                       o_vattr.at[pl.ds((va0 + 189) * 1024, 7 * 1024)],
                       sem_st2).wait()

    @pl.when(wid >= 10)
    def _():
      ho = va_fill(1, va0 + 189, 6)
      pltpu.async_copy(pat_f.at[pl.ds(ho, 6 * 1024)],
                       o_vattr.at[pl.ds((va0 + 189) * 1024, 6 * 1024)],
                       sem_st2).wait()

    wait_words(sem_st, 9216)

    @pl.when(wid < NV)
    def _():
      pltpu.sync_copy(wvn.at[pl.ds(wid * 128, 128)], pat_f.at[pl.ds(0, 128)])
      vs = [pat_f[pl.ds(r * 16, 16)] for r in range(8)]

      def repl(i, carry):
        for r in range(8):
          pat_f[pl.ds(i * 128 + r * 16, 16)] = vs[r]
        return carry

      lax.fori_loop(1, VH_W // 128, repl, 0)
      pltpu.async_copy(pat_f.at[pl.ds(0, VH_W)],
                       o_vh.at[pl.ds(wid * VH_W, VH_W)], sem_pat).wait()

    IDXW = 2 * E_BASE // NW
    ibbs = [db, stage]
    njobs = IDXW // IRING

    def istart(i):
      return pltpu.async_copy(
          ei.at[pl.ds(wid * IDXW + i * IRING, IRING)],
          ibbs[i % 2].at[pl.ds(0, IRING)],
          sem_in0 if i % 2 == 0 else sem_in1)

    i_in = [None] * njobs
    i_out = [None] * njobs
    i_in[0] = istart(0)
    for i in range(njobs):
      if i + 1 < njobs:
        if i >= 1:
          i_out[i - 1].wait()
        i_in[i + 1] = istart(i + 1)
      i_in[i].wait()
      i_out[i] = pltpu.async_copy(
          ibbs[i % 2].at[pl.ds(0, IRING)],
          o_idx.at[pl.ds(wid * IDXW + i * IRING, IRING)],
          sem_out0 if i % 2 == 0 else sem_out1)
    i_out[njobs - 2].wait()
    i_out[njobs - 1].wait()

    PERT = CB_BASE * 1024 // NW
    NCH = PERT // RING
    NJ = 2 * NCH

    def asrc(i):
      return (i // NCH) * (CB_BASE * 1024) + wid * PERT + (i % NCH) * RING

    def adst(i):
      return (i // NCH) * (CB_ALL * 1024) + wid * PERT + (i % NCH) * RING

    def await_ring(sem, buf):
      pltpu.make_async_copy(ea.at[pl.ds(0, RING)], buf, sem).wait()

    pltpu.async_copy(ea.at[pl.ds(asrc(0), RING)], bb0, sem_in0)

    def ring_body(i, carry):
      @pl.when(i + 1 < NJ)
      def _():
        @pl.when(i >= 1)
        def _():
          @pl.when((i + 1) % 2 == 0)
          def _():
            await_ring(sem_out0, bb0)

          @pl.when((i + 1) % 2 == 1)
          def _():
            await_ring(sem_out1, bb1)

        @pl.when((i + 1) % 2 == 0)
        def _():
          pltpu.async_copy(ea.at[pl.ds(asrc(i + 1), RING)], bb0, sem_in0)

        @pl.when((i + 1) % 2 == 1)
        def _():
          pltpu.async_copy(ea.at[pl.ds(asrc(i + 1), RING)], bb1, sem_in1)

      @pl.when(i % 2 == 0)
      def _():
        await_ring(sem_in0, bb0)
        pltpu.async_copy(bb0, o_attr.at[pl.ds(adst(i), RING)], sem_out0)

      @pl.when(i % 2 == 1)
      def _():
        await_ring(sem_in1, bb1)
        pltpu.async_copy(bb1, o_attr.at[pl.ds(adst(i), RING)], sem_out1)
      return carry

    lax.fori_loop(0, NJ, ring_body, 0)
    await_ring(sem_out0, bb0)
    await_ring(sem_out1, bb1)

    for d in drain:
      d.wait()

  return body(ei, ea, bv, ee, wfv, wvn, aux16)


def kernel(edge_index, edge_attr, batch_vec, expander_edges, num_graphs,
           exp_edge_attr_weight, virt_node_emb_weight,
           virt_edge_in_emb_weight, virt_edge_out_emb_weight):
  E_BASE = edge_index.shape[1]
  E_EXP = expander_edges.shape[0]
  N = batch_vec.shape[0]
  NV = virt_node_emb_weight.shape[0]

  ei = edge_index.reshape(2, E_BASE // 128, 128).transpose(1, 0, 2).reshape(-1)
  ea = (edge_attr.T.reshape(2, 8, E_BASE // 128, 128)
        .transpose(0, 2, 1, 3).reshape(-1))
  ee = expander_edges.T.reshape(-1)
  wseg = jnp.stack(
      [virt_edge_in_emb_weight, virt_edge_out_emb_weight], axis=1).reshape(-1)
  wfv = jnp.concatenate([exp_edge_attr_weight.reshape(-1), wseg])
  wvn = virt_node_emb_weight.reshape(-1)
  c4 = N + jnp.arange(NV, dtype=jnp.int32) * num_graphs
  aux16 = jnp.concatenate([c4, jnp.zeros((16 - NV,), jnp.int32)])

  o_idx, o_attr, o_types, o_vh, o_vei, o_vattr = _sc_impl(
      E_BASE, E_EXP, N, ei, ea, batch_vec, ee, wfv, wvn, aux16)

  E_ALL = E_BASE + E_EXP
  NVC = 2 * NV * N
  return (
      o_idx.reshape(E_ALL // 128, 2, 128).transpose(1, 0, 2).reshape(2, E_ALL),
      (o_attr.reshape(2, E_ALL // 128, 8, 128).transpose(0, 2, 1, 3)
       .reshape(16, E_ALL).T),
      o_types,
      o_vh.reshape(NV * G_STATIC, 128),
      o_vei.reshape(NVC // 128, 2, 128).transpose(1, 0, 2).reshape(2, NVC),
      (o_vattr.reshape(2, NVC // 128, 8, 128).transpose(0, 2, 1, 3)
       .reshape(16, NVC).T),
  )


# speedup vs baseline: 24.5960x; 1.0359x over previous
import functools

import jax
import jax.numpy as jnp
from jax import lax
from jax.experimental import pallas as pl
from jax.experimental.pallas import tpu as pltpu
from jax.experimental.pallas import tpu_sc as plsc

NC = 2
NS = 16
NW = NC * NS

G_STATIC = 128


def _sc_impl(E_BASE, E_EXP, N, ei, ea, bv, ee, wfv, wvn, aux16):
  NV = 4
  CB_BASE = E_BASE // 128
  CB_ALL = (E_BASE + E_EXP) // 128
  CB_EXP = CB_ALL - CB_BASE
  CB_VEI = 2 * NV * N // 128
  PAT = 19456
  RING = 10000
  IRING = 10000
  T0_PER = E_BASE // NW
  T1_PER = E_EXP // NW
  NT2 = 2 * NV * N
  T2_CHUNK = ((NT2 + NW - 1) // NW + 7) // 8 * 8
  T2_NFULL = NT2 // T2_CHUNK
  T2_REM = NT2 - T2_NFULL * T2_CHUNK
  VH_W = G_STATIC * 128

  O_IDX = 2 * (E_BASE + E_EXP)
  O_ATTR = (E_BASE + E_EXP) * 16
  O_TYPES = E_BASE + E_EXP + 2 * NV * N
  O_VEI = 2 * (2 * NV * N)
  O_VATTR = 2 * NV * N * 16

  mesh = plsc.VectorSubcoreMesh(
      core_axis_name="c", subcore_axis_name="s", num_cores=NC, num_subcores=NS)

  @functools.partial(
      pl.kernel,
      out_type=(
          jax.ShapeDtypeStruct((O_IDX,), jnp.int32),
          jax.ShapeDtypeStruct((O_ATTR,), jnp.float32),
          jax.ShapeDtypeStruct((O_TYPES,), jnp.int32),
          jax.ShapeDtypeStruct((NV * VH_W,), jnp.float32),
          jax.ShapeDtypeStruct((O_VEI,), jnp.int32),
          jax.ShapeDtypeStruct((O_VATTR,), jnp.float32),
      ),
      mesh=mesh,
      compiler_params=pltpu.CompilerParams(needs_layout_passes=False),
      scratch_types=[
          pltpu.VMEM((PAT,), jnp.float32),
          pltpu.VMEM((5488,), jnp.int32),
          pltpu.VMEM((50000,), jnp.int32),
          pltpu.VMEM((12544,), jnp.int32),
          pltpu.VMEM((12544,), jnp.int32),
          pltpu.VMEM((RING,), jnp.float32),
          pltpu.VMEM((RING,), jnp.float32),
          pltpu.VMEM((RING,), jnp.float32),
          pltpu.VMEM((16,), jnp.int32),
          pltpu.VMEM((144,), jnp.float32),
          pltpu.SemaphoreType.DMA,
          pltpu.SemaphoreType.DMA,
          pltpu.SemaphoreType.DMA,
          pltpu.SemaphoreType.DMA,
          pltpu.SemaphoreType.DMA,
          pltpu.SemaphoreType.DMA,
          pltpu.SemaphoreType.DMA,
          pltpu.SemaphoreType.DMA,
          pltpu.SemaphoreType.DMA,
          pltpu.SemaphoreType.DMA,
      ],
  )
  def body(ei, ea, bv, ee, wfv, wvn, aux16,
           o_idx, o_attr, o_types, o_vh, o_vei, o_vattr,
           pat_f, cbuf, bv_v, stage, db, bb0, bb1, bb2, auxv, wfv_v,
           sem_pat, sem_bg, sem_st, sem_st2, sem_in0, sem_in1,
           sem_out0, sem_out1, sem_in2, sem_out2):
    wid = lax.axis_index("s") * NC + lax.axis_index("c")
    drain = []
    iota = lax.iota(jnp.int32, 16)
    zeros16 = jnp.zeros((16,), jnp.int32)

    pltpu.sync_copy(wfv, wfv_v)
    pltpu.sync_copy(aux16, auxv)

    rb = wid % 2
    w = wid // 2
    for r in range(8):
      vec = plsc.load_gather(wfv_v, [zeros16 + (rb * 8 + r)])

      def fillp(t, carry):
        for v in range(8):
          pat_f[pl.ds(t * 1024 + r * 128 + v * 16, 16)] = vec
        return carry
      lax.fori_loop(0, PAT // 1024, fillp, 0)
    exp_w0 = rb * CB_ALL * 1024 + CB_BASE * 1024
    tile0 = jnp.where(w < 10, w * 391, 3910 + (w - 10) * 390)
    off = exp_w0 + tile0 * 1024
    p2_outs = []
    for j in range(20):
      p2_outs.append(pltpu.async_copy(
          pat_f, o_attr.at[pl.ds(off + j * PAT, PAT)], sem_pat))

    @pl.when(w < 10)
    def _():
      pltpu.async_copy(pat_f.at[pl.ds(0, 11264)],
                       o_attr.at[pl.ds(off + 20 * PAT, 11264)],
                       sem_pat).wait()

    @pl.when(w >= 10)
    def _():
      pltpu.async_copy(pat_f.at[pl.ds(0, 10240)],
                       o_attr.at[pl.ds(off + 20 * PAT, 10240)],
                       sem_pat).wait()

    def cfill(buf, val, start, nv):
      vec = zeros16 + val

      def bodyf(i, carry):
        buf[pl.ds(i * 16, 16)] = vec
        return carry
      lax.fori_loop(start, nv, bodyf, 0)

    cfill(cbuf, 0, 0, 156)
    cfill(cbuf, 1, 156, 281)
    cfill(cbuf, 2, 281, 343)
    base = wid * T0_PER
    for j in range(20):
      drain.append(pltpu.async_copy(
          cbuf.at[pl.ds(0, 2496)],
          o_types.at[pl.ds(base + j * 2496, 2496)], sem_bg))
    drain.append(pltpu.async_copy(
        cbuf.at[pl.ds(0, 80)], o_types.at[pl.ds(base + 49920, 80)], sem_bg))
    base = E_BASE + wid * T1_PER
    for j in range(12):
      drain.append(pltpu.async_copy(
          cbuf.at[pl.ds(2496, 2000)],
          o_types.at[pl.ds(base + j * 2000, 2000)], sem_bg))
    drain.append(pltpu.async_copy(
        cbuf.at[pl.ds(2496, 1000)],
        o_types.at[pl.ds(base + 24000, 1000)], sem_bg))
    base = E_BASE + E_EXP

    def t2_copies(b2, tail):
      for j in range(12):
        pltpu.async_copy(cbuf.at[pl.ds(4496, 992)],
                         o_types.at[pl.ds(b2 + j * 992, 992)], sem_bg).wait()
      pltpu.async_copy(cbuf.at[pl.ds(4496, tail)],
                       o_types.at[pl.ds(b2 + 11904, tail)], sem_bg).wait()

    @pl.when(wid < T2_NFULL)
    def _():
      t2_copies(base + wid * T2_CHUNK, T2_CHUNK - 11904)

    @pl.when(wid == T2_NFULL)
    def _():
      t2_copies(base + T2_NFULL * T2_CHUNK, T2_REM - 11904)

    pltpu.sync_copy(bv, bv_v)
    vb0 = jnp.where(wid < 21, wid * 98, 2058 + (wid - 21) * 97)

    def make_vei_block(buf, cb0):
      def vei_block(b, carry):
        cb = cb0 + b
        for v in range(8):
          colv = iota + (cb * 128 + v * 16)
          s = colv // N
          il = colv - s * N
          bvv = plsc.load_gather(bv_v, [il])
          cv = plsc.load_gather(auxv, [s >> 1])
          vt = bvv + cv
          evn = (s & 1) == 0
          buf[pl.ds(b * 256 + v * 16, 16)] = jnp.where(evn, il, vt)
          buf[pl.ds(b * 256 + 128 + v * 16, 16)] = jnp.where(evn, vt, il)
        return carry
      return vei_block

    lax.fori_loop(0, 49, make_vei_block(stage, vb0), 0)
    vei_a = pltpu.async_copy(stage.at[pl.ds(0, 49 * 256)],
                             o_vei.at[pl.ds(vb0 * 256, 49 * 256)], sem_st)
    nblk = jnp.where(wid < 21, 98, 97)
    lax.fori_loop(0, nblk - 49, make_vei_block(db, vb0 + 49), 0)

    @pl.when(wid < 21)
    def _():
      pltpu.async_copy(db.at[pl.ds(0, 49 * 256)],
                       o_vei.at[pl.ds((vb0 + 49) * 256, 49 * 256)],
                       sem_st2).wait()

    @pl.when(wid >= 21)
    def _():
      pltpu.async_copy(db.at[pl.ds(0, 48 * 256)],
                       o_vei.at[pl.ds((vb0 + 49) * 256, 48 * 256)],
                       sem_st2).wait()
    vei_a.wait()

    ib0 = jnp.where(wid < 10, wid * 196, 1960 + (wid - 10) * 195)

    def idx_round(blk0, nb):
      pltpu.sync_copy(ee.at[pl.ds(blk0 * 128, nb * 128)],
                      db.at[pl.ds(0, nb * 128)])
      pltpu.sync_copy(ee.at[pl.ds(E_EXP + blk0 * 128, nb * 128)],
                      db.at[pl.ds(6272, nb * 128)])

      def inter(b, carry):
        for v in range(8):
          stage[pl.ds(b * 256 + v * 16, 16)] = db[pl.ds(b * 128 + v * 16, 16)]
          stage[pl.ds(b * 256 + 128 + v * 16, 16)] = (
              db[pl.ds(6272 + b * 128 + v * 16, 16)])
        return carry

      lax.fori_loop(0, nb, inter, 0)
      pltpu.async_copy(
          stage.at[pl.ds(0, nb * 256)],
          o_idx.at[pl.ds((CB_BASE + blk0) * 256, nb * 256)], sem_st).wait()

    for rr in range(3):
      idx_round(ib0 + rr * 49, 49)

    @pl.when(wid < 10)
    def _():
      idx_round(ib0 + 147, 49)

    @pl.when(wid >= 10)
    def _():
      idx_round(ib0 + 147, 48)

    for d in p2_outs:
      d.wait()
    va0 = jnp.where(wid < 10, wid * 196, 1960 + (wid - 10) * 195)
    vhalves = [0, 9728]

    def wait_words(sem, nwords):
      pltpu.make_async_copy(o_vattr.at[pl.ds(0, nwords)],
                            pat_f.at[pl.ds(0, nwords)], sem).wait()

    def va_fill(rnd, t0, nt):
      half_off = (rnd % 2) * 9728

      def vat(t, carry):
        T = t0 + t
        rb2 = T // CB_VEI
        cb2 = T % CB_VEI
        s0 = cb2 * 128 // N
        s127 = (cb2 * 128 + 127) // N

        @pl.when(s0 == s127)
        def _():
          for r in range(8):
            val = plsc.load_gather(
                wfv_v, [zeros16 + (s0 * 16 + 16 + rb2 * 8 + r)])
            for v in range(8):
              pat_f[pl.ds(half_off + t * 1024 + r * 128 + v * 16, 16)] = val

        @pl.when(s0 != s127)
        def _():
          for r in range(8):
            for v in range(8):
              colv = iota + (cb2 * 128 + v * 16)
              s = colv // N
              val = plsc.load_gather(wfv_v, [s * 16 + (16 + rb2 * 8 + r)])
              pat_f[pl.ds(half_off + t * 1024 + r * 128 + v * 16, 16)] = val
        return carry

      lax.fori_loop(0, nt, vat, 0)
      return half_off

    def va_body(rr, carry):
      t0 = va0 + rr * 9

      @pl.when(rr >= 2)
      def _():
        @pl.when(rr % 2 == 0)
        def _():
          wait_words(sem_st, 9216)

        @pl.when(rr % 2 == 1)
        def _():
          wait_words(sem_st2, 9216)

      va_fill(rr, t0, 9)

      @pl.when(rr % 2 == 0)
      def _():
        pltpu.async_copy(pat_f.at[pl.ds(0, 9216)],
                         o_vattr.at[pl.ds(t0 * 1024, 9216)], sem_st)

      @pl.when(rr % 2 == 1)
      def _():
        pltpu.async_copy(pat_f.at[pl.ds(9728, 9216)],
                         o_vattr.at[pl.ds(t0 * 1024, 9216)], sem_st2)
      return carry

    lax.fori_loop(0, 21, va_body, 0)
    wait_words(sem_st2, 9216)

    @pl.when(wid < 10)
    def _():
      ho = va_fill(1, va0 + 189, 7)
      pltpu.async_copy(pat_f.at[pl.ds(ho, 7 * 1024)],
                       o_vattr.at[pl.ds((va0 + 189) * 1024, 7 * 1024)],
                       sem_st2).wait()

    @pl.when(wid >= 10)
    def _():
      ho = va_fill(1, va0 + 189, 6)
      pltpu.async_copy(pat_f.at[pl.ds(ho, 6 * 1024)],
                       o_vattr.at[pl.ds((va0 + 189) * 1024, 6 * 1024)],
                       sem_st2).wait()

    wait_words(sem_st, 9216)

    @pl.when(wid < NV)
    def _():
      pltpu.sync_copy(wvn.at[pl.ds(wid * 128, 128)], pat_f.at[pl.ds(0, 128)])
      vs = [pat_f[pl.ds(r * 16, 16)] for r in range(8)]

      def repl(i, carry):
        for r in range(8):
          pat_f[pl.ds(i * 128 + r * 16, 16)] = vs[r]
        return carry

      lax.fori_loop(1, VH_W // 128, repl, 0)
      pltpu.async_copy(pat_f.at[pl.ds(0, VH_W)],
                       o_vh.at[pl.ds(wid * VH_W, VH_W)], sem_pat).wait()

    IDXW = 2 * E_BASE // NW
    ibbs = [db, stage]
    njobs = IDXW // IRING

    def istart(i):
      return pltpu.async_copy(
          ei.at[pl.ds(wid * IDXW + i * IRING, IRING)],
          ibbs[i % 2].at[pl.ds(0, IRING)],
          sem_in0 if i % 2 == 0 else sem_in1)

    i_in = [None] * njobs
    i_out = [None] * njobs
    i_in[0] = istart(0)
    for i in range(njobs):
      if i + 1 < njobs:
        if i >= 1:
          i_out[i - 1].wait()
        i_in[i + 1] = istart(i + 1)
      i_in[i].wait()
      i_out[i] = pltpu.async_copy(
          ibbs[i % 2].at[pl.ds(0, IRING)],
          o_idx.at[pl.ds(wid * IDXW + i * IRING, IRING)],
          sem_out0 if i % 2 == 0 else sem_out1)
    i_out[njobs - 2].wait()
    i_out[njobs - 1].wait()

    PERT = CB_BASE * 1024 // NW
    NCH = PERT // RING
    NJ = 2 * NCH

    def asrc(i):
      return (i // NCH) * (CB_BASE * 1024) + wid * PERT + (i % NCH) * RING

    def adst(i):
      return (i // NCH) * (CB_ALL * 1024) + wid * PERT + (i % NCH) * RING

    def await_ring(sem, buf):
      pltpu.make_async_copy(ea.at[pl.ds(0, RING)], buf, sem).wait()

    rbufs = [bb0, bb1, bb2]
    rins = [sem_in0, sem_in1, sem_in2]
    routs = [sem_out0, sem_out1, sem_out2]
    pltpu.async_copy(ea.at[pl.ds(asrc(0), RING)], bb0, sem_in0)
    pltpu.async_copy(ea.at[pl.ds(asrc(1), RING)], bb1, sem_in1)

    def ring_body(i, carry):
      @pl.when(i + 2 < NJ)
      def _():
        for m in range(3):
          @pl.when((i + 2) % 3 == m)
          def _():
            @pl.when(i >= 1)
            def _():
              await_ring(routs[m], rbufs[m])
            pltpu.async_copy(ea.at[pl.ds(asrc(i + 2), RING)],
                             rbufs[m], rins[m])

      for m in range(3):
        @pl.when(i % 3 == m)
        def _():
          await_ring(rins[m], rbufs[m])
          pltpu.async_copy(rbufs[m], o_attr.at[pl.ds(adst(i), RING)],
                           routs[m])
      return carry

    lax.fori_loop(0, NJ, ring_body, 0)
    for m in ((NJ - 3) % 3, (NJ - 2) % 3, (NJ - 1) % 3):
      await_ring(routs[m], rbufs[m])

    for d in drain:
      d.wait()

  return body(ei, ea, bv, ee, wfv, wvn, aux16)


def kernel(edge_index, edge_attr, batch_vec, expander_edges, num_graphs,
           exp_edge_attr_weight, virt_node_emb_weight,
           virt_edge_in_emb_weight, virt_edge_out_emb_weight):
  E_BASE = edge_index.shape[1]
  E_EXP = expander_edges.shape[0]
  N = batch_vec.shape[0]
  NV = virt_node_emb_weight.shape[0]

  ei = edge_index.reshape(2, E_BASE // 128, 128).transpose(1, 0, 2).reshape(-1)
  ea = (edge_attr.T.reshape(2, 8, E_BASE // 128, 128)
        .transpose(0, 2, 1, 3).reshape(-1))
  ee = expander_edges.T.reshape(-1)
  wseg = jnp.stack(
      [virt_edge_in_emb_weight, virt_edge_out_emb_weight], axis=1).reshape(-1)
  wfv = jnp.concatenate([exp_edge_attr_weight.reshape(-1), wseg])
  wvn = virt_node_emb_weight.reshape(-1)
  c4 = N + jnp.arange(NV, dtype=jnp.int32) * num_graphs
  aux16 = jnp.concatenate([c4, jnp.zeros((16 - NV,), jnp.int32)])

  o_idx, o_attr, o_types, o_vh, o_vei, o_vattr = _sc_impl(
      E_BASE, E_EXP, N, ei, ea, batch_vec, ee, wfv, wvn, aux16)

  E_ALL = E_BASE + E_EXP
  NVC = 2 * NV * N
  return (
      o_idx.reshape(E_ALL // 128, 2, 128).transpose(1, 0, 2).reshape(2, E_ALL),
      (o_attr.reshape(2, E_ALL // 128, 8, 128).transpose(0, 2, 1, 3)
       .reshape(16, E_ALL).T),
      o_types,
      o_vh.reshape(NV * G_STATIC, 128),
      o_vei.reshape(NVC // 128, 2, 128).transpose(1, 0, 2).reshape(2, NVC),
      (o_vattr.reshape(2, NVC // 128, 8, 128).transpose(0, 2, 1, 3)
       .reshape(16, NVC).T),
  )
